# baseline (speedup 1.0000x reference)
import functools

import jax
import jax.numpy as jnp
from jax import lax
from jax.experimental import pallas as pl
from jax.experimental.pallas import tpu as pltpu
from jax.experimental.pallas import tpu_sc as plsc

DIM = 128
PROVIDER_DIM = 64
VOCAB = 20
NUM_TRUST = 6
B, N = 4096, 50
R = B * N
OUT_ROWS = 3 * R
NC, NS = 2, 16
NW = NC * NS
ORPW = OUT_ROWS // NW
GC = 128
NSLOT = 6
NBODY = ORPW // (GC * NSLOT)

_mesh = plsc.VectorSubcoreMesh(
    core_axis_name="c", subcore_axis_name="s", num_cores=NC, num_subcores=NS)


@functools.partial(
    pl.kernel,
    out_type=jax.ShapeDtypeStruct((OUT_ROWS, DIM), jnp.float32),
    mesh=_mesh,
    scratch_types=(
        [pltpu.VMEM((ORPW,), jnp.int32)]
        + [pltpu.VMEM((GC, DIM), jnp.float32)] * NSLOT
        + [pltpu.SemaphoreType.DMA] * (2 * NSLOT)
    ),
)
def _encode(g_hbm, tab_hbm, out_hbm, g_v, *bufs_and_sems):
    bufs = bufs_and_sems[:NSLOT]
    gsems = bufs_and_sems[NSLOT:2 * NSLOT]
    osems = bufs_and_sems[2 * NSLOT:]

    wid = lax.axis_index("s") * NC + lax.axis_index("c")
    base = wid * ORPW
    pltpu.sync_copy(g_hbm.at[pl.ds(base, ORPW)], g_v)

    def fire_gather(k, j):
        sl = pl.ds((NSLOT * k + j) * GC, GC)
        return pltpu.async_copy(tab_hbm.at[g_v.at[sl]], bufs[j], gsems[j])

    def fire_write(k, j):
        dst = out_hbm.at[pl.ds(base + (NSLOT * k + j) * GC, GC)]
        return pltpu.async_copy(bufs[j], dst, osems[j])

    def wait_write(j):
        pltpu.make_async_copy(bufs[j], out_hbm.at[pl.ds(0, GC)], osems[j]).wait()

    def body(k, carry):
        @pl.when(k > 0)
        def _():
            for j in range(NSLOT):
                wait_write(j)

        handles = [fire_gather(k, j) for j in range(NSLOT)]
        for j in range(NSLOT):
            handles[j].wait()
            fire_write(k, j)
        return carry

    lax.fori_loop(0, NBODY, body, 0)
    for j in range(NSLOT):
        wait_write(j)


def kernel(node_types, trust_levels, providers, node_attrs, type_table,
           trust_table, provider_table, attr_W, attr_b):
    del node_attrs, attr_W, attr_b
    t = node_types.astype(jnp.int32).T
    r = trust_levels.astype(jnp.int32).T
    p = providers.astype(jnp.int32).T
    g = jnp.stack([t, r + VOCAB, p + (VOCAB + NUM_TRUST)], axis=-1).reshape(OUT_ROWS)
    tab = jnp.concatenate([
        type_table,
        trust_table,
        jnp.pad(provider_table, ((0, 0), (0, DIM - PROVIDER_DIM))),
    ], axis=0)
    out = _encode(g, tab)
    return out.reshape(N, B, 3 * DIM).transpose(1, 0, 2)


# speedup vs baseline: 10.0631x; 10.0631x over previous
import functools

import jax
import jax.numpy as jnp
from jax import lax
from jax.experimental import pallas as pl
from jax.experimental.pallas import tpu as pltpu
from jax.experimental.pallas import tpu_sc as plsc

DIM = 128
PROVIDER_DIM = 64
VOCAB = 20
NUM_TRUST = 6
NUM_PROV = 16
B, N = 4096, 50
R = B * N
ODIM = 3 * DIM
NC, NS = 2, 16
NW = NC * NS
RPW = R // NW
GC = 128
NSLOT = 2
NBODY = RPW // (GC * NSLOT)

_mesh = plsc.VectorSubcoreMesh(
    core_axis_name="c", subcore_axis_name="s", num_cores=NC, num_subcores=NS)


@functools.partial(
    pl.kernel,
    out_type=jax.ShapeDtypeStruct((R, ODIM), jnp.float32),
    mesh=_mesh,
    scratch_types=(
        [pltpu.VMEM((RPW,), jnp.int32)]
        + [pltpu.VMEM((GC, ODIM), jnp.float32)] * NSLOT
        + [pltpu.SemaphoreType.DMA] * (2 * NSLOT)
    ),
)
def _encode(f_hbm, tab_hbm, out_hbm, f_v, *bufs_and_sems):
    bufs = bufs_and_sems[:NSLOT]
    gsems = bufs_and_sems[NSLOT:2 * NSLOT]
    osems = bufs_and_sems[2 * NSLOT:]

    wid = lax.axis_index("s") * NC + lax.axis_index("c")
    base = wid * RPW
    pltpu.sync_copy(f_hbm.at[pl.ds(base, RPW)], f_v)

    def fire_gather(k, j):
        sl = pl.ds((NSLOT * k + j) * GC, GC)
        return pltpu.async_copy(tab_hbm.at[f_v.at[sl]], bufs[j], gsems[j])

    def fire_write(k, j):
        dst = out_hbm.at[pl.ds(base + (NSLOT * k + j) * GC, GC)]
        return pltpu.async_copy(bufs[j], dst, osems[j])

    def wait_write(j):
        pltpu.make_async_copy(bufs[j], out_hbm.at[pl.ds(0, GC)], osems[j]).wait()

    def body(k, carry):
        @pl.when(k > 0)
        def _():
            for j in range(NSLOT):
                wait_write(j)

        handles = [fire_gather(k, j) for j in range(NSLOT)]
        for j in range(NSLOT):
            handles[j].wait()
            fire_write(k, j)
        return carry

    lax.fori_loop(0, NBODY, body, 0)
    for j in range(NSLOT):
        wait_write(j)


def kernel(node_types, trust_levels, providers, node_attrs, type_table,
           trust_table, provider_table, attr_W, attr_b):
    del node_attrs, attr_W, attr_b
    t = node_types.astype(jnp.int32).T
    r = trust_levels.astype(jnp.int32).T
    p = providers.astype(jnp.int32).T
    f = ((t * NUM_TRUST + r) * NUM_PROV + p).reshape(R)
    shape4 = (VOCAB, NUM_TRUST, NUM_PROV, DIM)
    tpart = jnp.broadcast_to(type_table[:, None, None, :], shape4)
    rpart = jnp.broadcast_to(trust_table[None, :, None, :], shape4)
    ppad = jnp.pad(provider_table, ((0, 0), (0, DIM - PROVIDER_DIM)))
    ppart = jnp.broadcast_to(ppad[None, None, :, :], shape4)
    tab = jnp.concatenate([tpart, rpart, ppart], axis=-1).reshape(
        VOCAB * NUM_TRUST * NUM_PROV, ODIM)
    out = _encode(f, tab)
    return out.reshape(N, B, ODIM).transpose(1, 0, 2)


# speedup vs baseline: 10.1046x; 1.0041x over previous
import functools

import jax
import jax.numpy as jnp
from jax import lax
from jax.experimental import pallas as pl
from jax.experimental.pallas import tpu as pltpu
from jax.experimental.pallas import tpu_sc as plsc

DIM = 128
PROVIDER_DIM = 64
VOCAB = 20
NUM_TRUST = 6
NUM_PROV = 16
B, N = 4096, 50
R = B * N
ODIM = 3 * DIM
NC, NS = 2, 16
NW = NC * NS
RPW = R // NW
GC = 64
NSLOT = 5
NBODY = RPW // (GC * NSLOT)

_mesh = plsc.VectorSubcoreMesh(
    core_axis_name="c", subcore_axis_name="s", num_cores=NC, num_subcores=NS)


@functools.partial(
    pl.kernel,
    out_type=jax.ShapeDtypeStruct((R, ODIM), jnp.float32),
    mesh=_mesh,
    scratch_types=(
        [pltpu.VMEM((RPW,), jnp.int32)]
        + [pltpu.VMEM((GC, ODIM), jnp.float32)] * NSLOT
        + [pltpu.SemaphoreType.DMA] * (2 * NSLOT)
    ),
)
def _encode(f_hbm, tab_hbm, out_hbm, f_v, *bufs_and_sems):
    bufs = bufs_and_sems[:NSLOT]
    gsems = bufs_and_sems[NSLOT:2 * NSLOT]
    osems = bufs_and_sems[2 * NSLOT:]

    wid = lax.axis_index("s") * NC + lax.axis_index("c")
    base = wid * RPW
    pltpu.sync_copy(f_hbm.at[pl.ds(base, RPW)], f_v)

    def fire_gather(k, j):
        sl = pl.ds((NSLOT * k + j) * GC, GC)
        return pltpu.async_copy(tab_hbm.at[f_v.at[sl]], bufs[j], gsems[j])

    def fire_write(k, j):
        dst = out_hbm.at[pl.ds(base + (NSLOT * k + j) * GC, GC)]
        return pltpu.async_copy(bufs[j], dst, osems[j])

    def wait_write(j):
        pltpu.make_async_copy(bufs[j], out_hbm.at[pl.ds(0, GC)], osems[j]).wait()

    def body(k, carry):
        @pl.when(k > 0)
        def _():
            for j in range(NSLOT):
                wait_write(j)

        handles = [fire_gather(k, j) for j in range(NSLOT)]
        for j in range(NSLOT):
            handles[j].wait()
            fire_write(k, j)
        return carry

    lax.fori_loop(0, NBODY, body, 0)
    for j in range(NSLOT):
        wait_write(j)


def kernel(node_types, trust_levels, providers, node_attrs, type_table,
           trust_table, provider_table, attr_W, attr_b):
    del node_attrs, attr_W, attr_b
    t = node_types.astype(jnp.int32).T
    r = trust_levels.astype(jnp.int32).T
    p = providers.astype(jnp.int32).T
    f = ((t * NUM_TRUST + r) * NUM_PROV + p).reshape(R)
    shape4 = (VOCAB, NUM_TRUST, NUM_PROV, DIM)
    tpart = jnp.broadcast_to(type_table[:, None, None, :], shape4)
    rpart = jnp.broadcast_to(trust_table[None, :, None, :], shape4)
    ppad = jnp.pad(provider_table, ((0, 0), (0, DIM - PROVIDER_DIM)))
    ppart = jnp.broadcast_to(ppad[None, None, :, :], shape4)
    tab = jnp.concatenate([tpart, rpart, ppart], axis=-1).reshape(
        VOCAB * NUM_TRUST * NUM_PROV, ODIM)
    out = _encode(f, tab)
    return out.reshape(N, B, ODIM).transpose(1, 0, 2)
